# Initial kernel scaffold; baseline (speedup 1.0000x reference)
import jax
import jax.numpy as jnp
from jax.experimental import pallas as pl


def kernel(x, edge_index, batch, W1, b1, W2, b2, W3, b3, bn1_g, bn1_b, bn1_m, bn1_v, bn2_g, bn2_b, bn2_m, bn2_v, bn3_g, bn3_b, bn3_m, bn3_v, Wl, bl):
    raise NotImplementedError("write your pallas kernel here")


# speedup vs baseline: 1.4310x; 1.4310x over previous
import functools

import jax
import jax.numpy as jnp
from jax import lax
from jax.experimental import pallas as pl
from jax.experimental.pallas import tpu as pltpu
from jax.experimental.pallas import tpu_sc as plsc

N = 10000
E = 160000
H = 512
G = 64
LANES = 16
NWORK = 32
BLK = 160
NBLK = 63
NPAD = NBLK * BLK
EA = E + N
VPR = H // LANES
MGRID = 9
MB = NPAD // MGRID


def _mm_body(x_ref, w_ref, o_ref):
    o_ref[...] = jnp.dot(x_ref[...], w_ref[...],
                         preferred_element_type=jnp.float32,
                         precision=lax.Precision.HIGHEST)


def _matmul(x, w):
    k = x.shape[1]
    no = w.shape[1]
    return pl.pallas_call(
        _mm_body,
        grid=(MGRID,),
        in_specs=[pl.BlockSpec((MB, k), lambda i: (i, 0)),
                  pl.BlockSpec((k, no), lambda i: (0, 0))],
        out_specs=pl.BlockSpec((MB, no), lambda i: (i, 0)),
        out_shape=jax.ShapeDtypeStruct((NPAD, no), jnp.float32),
    )(x, w)


def _agg_body(h_hbm, src_hbm, dst_hbm, nrm_hbm, offs_hbm, sc_hbm, sh_hbm,
              out_hbm,
              offrow_v, idx_v, dstv_v, nrm_v, rows_v, acc_v, scale_v,
              shift_v, sem):
    wid = lax.axis_index("s") * 2 + lax.axis_index("c")
    pltpu.sync_copy(sc_hbm, scale_v)
    pltpu.sync_copy(sh_hbm, shift_v)
    lane = lax.iota(jnp.int32, LANES)

    for t in range(2):
        b = wid + t * NWORK

        @pl.when(b < NBLK)
        def _block():
            base_node = b * BLK

            def zero_body(r, carry):
                for v in range(VPR):
                    acc_v[r, pl.ds(v * LANES, LANES)] = jnp.zeros(
                        (LANES,), jnp.float32)
                return carry
            lax.fori_loop(0, BLK, zero_body, 0)

            pltpu.sync_copy(offs_hbm.at[pl.ds(b * LANES, LANES)], offrow_v)
            offrow = offrow_v[...]
            e0 = offrow[0]
            e1 = offrow[1]
            b0 = (e0 // LANES) * LANES
            nbat = (e1 - b0 + LANES - 1) // LANES

            def batch_body(kk, carry):
                base = b0 + kk * LANES
                pltpu.sync_copy(src_hbm.at[pl.ds(base, LANES)], idx_v)
                pltpu.sync_copy(dst_hbm.at[pl.ds(base, LANES)], dstv_v)
                pltpu.sync_copy(nrm_hbm.at[pl.ds(base, LANES)], nrm_v)
                pltpu.async_copy(h_hbm.at[idx_v], rows_v, sem).wait()
                gidx = base + lane
                valid = (gidx >= e0) & (gidx < e1)
                nrm = jnp.where(valid, nrm_v[...], 0.0)
                dstl = jnp.where(valid, dstv_v[...] - base_node, 0)
                for j in range(LANES):
                    njs = nrm[j]
                    djs = dstl[j]
                    njv = jnp.full((LANES,), njs, jnp.float32)
                    for v in range(VPR):
                        sl = pl.ds(v * LANES, LANES)
                        plsc.addupdate(acc_v.at[djs, sl],
                                       njv * rows_v[j, sl])
                return carry
            lax.fori_loop(0, nbat, batch_body, 0)

            def wb_body(r, carry):
                for v in range(VPR):
                    sl = pl.ds(v * LANES, LANES)
                    y = acc_v[r, sl] * scale_v[sl] + shift_v[sl]
                    acc_v[r, sl] = jnp.maximum(y, 0.0)
                return carry
            lax.fori_loop(0, BLK, wb_body, 0)

            pltpu.sync_copy(acc_v, out_hbm.at[pl.ds(base_node, BLK)])


def _aggregate(h, src_s, dst_s, nrm_s, offs, scale, shift):
    mesh = plsc.VectorSubcoreMesh(core_axis_name="c", subcore_axis_name="s")
    kfn = pl.kernel(
        _agg_body,
        out_type=jax.ShapeDtypeStruct((NPAD, H), jnp.float32),
        mesh=mesh,
        scratch_types=[
            pltpu.VMEM((LANES,), jnp.int32),
            pltpu.VMEM((LANES,), jnp.int32),
            pltpu.VMEM((LANES,), jnp.int32),
            pltpu.VMEM((LANES,), jnp.float32),
            pltpu.VMEM((LANES, H), jnp.float32),
            pltpu.VMEM((BLK, H), jnp.float32),
            pltpu.VMEM((H,), jnp.float32),
            pltpu.VMEM((H,), jnp.float32),
            pltpu.SemaphoreType.DMA,
        ],
    )
    return kfn(h, src_s, dst_s, nrm_s, offs, scale, shift)


def _pool_body(a_ref, b_ref, wl_ref, bl_ref, o_ref, sums, cnt):
    i = pl.program_id(0)

    @pl.when(i == 0)
    def _init():
        sums[...] = jnp.zeros_like(sums)
        cnt[...] = jnp.zeros_like(cnt)

    bv = b_ref[0]
    gids = lax.broadcasted_iota(jnp.int32, (G, MB), 0)
    m = jnp.where(bv == gids, 1.0, 0.0)
    sums[...] += jnp.dot(m, a_ref[...],
                         preferred_element_type=jnp.float32,
                         precision=lax.Precision.HIGHEST)
    cnt[...] += jnp.broadcast_to(jnp.sum(m, axis=1, keepdims=True), (G, 128))

    @pl.when(i == pl.num_programs(0) - 1)
    def _fin():
        c = jnp.maximum(cnt[:, 0:1], 1.0)
        pooled = sums[...] / c
        mu = jnp.mean(pooled, axis=-1, keepdims=True)
        var = jnp.mean((pooled - mu) ** 2, axis=-1, keepdims=True)
        ln = (pooled - mu) * lax.rsqrt(var + 1e-5)
        o_ref[...] = jnp.dot(ln, wl_ref[...],
                             preferred_element_type=jnp.float32,
                             precision=lax.Precision.HIGHEST) + bl_ref[...]


def _pool(a, batch3, wlp, blr):
    return pl.pallas_call(
        _pool_body,
        grid=(MGRID,),
        in_specs=[pl.BlockSpec((MB, H), lambda i: (i, 0)),
                  pl.BlockSpec((1, 1, MB), lambda i: (i, 0, 0)),
                  pl.BlockSpec((H, 128), lambda i: (0, 0)),
                  pl.BlockSpec((1, 128), lambda i: (0, 0))],
        out_specs=pl.BlockSpec((G, 128), lambda i: (0, 0)),
        out_shape=jax.ShapeDtypeStruct((G, 128), jnp.float32),
        scratch_shapes=[pltpu.VMEM((G, H), jnp.float32),
                        pltpu.VMEM((G, 128), jnp.float32)],
    )(a, batch3, wlp, blr)


def kernel(x, edge_index, batch, W1, b1, W2, b2, W3, b3,
           bn1_g, bn1_b, bn1_m, bn1_v,
           bn2_g, bn2_b, bn2_m, bn2_v,
           bn3_g, bn3_b, bn3_m, bn3_v,
           Wl, bl):
    f32 = jnp.float32
    ar = jnp.arange(N, dtype=jnp.int32)
    src_a = jnp.concatenate([edge_index[0], ar])
    dst_a = jnp.concatenate([edge_index[1], ar])
    perm = jnp.argsort(dst_a)
    src_s = src_a[perm]
    dst_s = dst_a[perm]
    left = jnp.searchsorted(dst_s, ar, side="left")
    right = jnp.searchsorted(dst_s, ar, side="right")
    deg = (right - left).astype(f32)
    dinv = lax.rsqrt(deg)
    nrm_s = dinv[src_s] * dinv[dst_s]
    bounds = jnp.arange(NBLK + 1, dtype=jnp.int32) * BLK
    off = jnp.searchsorted(dst_s, bounds, side="left").astype(jnp.int32)
    offs = (jnp.zeros((NBLK, LANES), jnp.int32)
            .at[:, 0].set(off[:-1])
            .at[:, 1].set(off[1:])).reshape(-1)

    def fold(g, bta, m, v, b_lin):
        sc = g * lax.rsqrt(v + 1e-5)
        return sc, (b_lin - m) * sc + bta

    sc1, sh1 = fold(bn1_g, bn1_b, bn1_m, bn1_v, b1)
    sc2, sh2 = fold(bn2_g, bn2_b, bn2_m, bn2_v, b2)
    sc3, sh3 = fold(bn3_g, bn3_b, bn3_m, bn3_v, b3)

    xp = jnp.pad(x, ((0, NPAD - N), (0, 0)))
    h = _matmul(xp, W1)
    a = _aggregate(h, src_s, dst_s, nrm_s, offs, sc1, sh1)
    h = _matmul(a, W2)
    a = _aggregate(h, src_s, dst_s, nrm_s, offs, sc2, sh2)
    h = _matmul(a, W3)
    a = _aggregate(h, src_s, dst_s, nrm_s, offs, sc3, sh3)

    batch3 = jnp.pad(batch, (0, NPAD - N), constant_values=G)
    batch3 = batch3.reshape(MGRID, 1, MB)
    wlp = jnp.pad(Wl, ((0, 0), (0, 127)))
    blr = jnp.broadcast_to(bl.reshape(1, 1), (1, 128))
    out = _pool(a, batch3, wlp, blr)
    return out[:, 0:1]


# speedup vs baseline: 1.7581x; 1.2286x over previous
import functools

import jax
import jax.numpy as jnp
from jax import lax
from jax.experimental import pallas as pl
from jax.experimental.pallas import tpu as pltpu
from jax.experimental.pallas import tpu_sc as plsc

N = 10000
E = 160000
H = 512
G = 64
LANES = 16
NWORK = 32
BLK = 160
NBLK = 63
NPAD = NBLK * BLK
EA = E + N
VPR = H // LANES
MGRID = 9
MB = NPAD // MGRID
NBT = EA // LANES
MC = 32
MROW = 2 * LANES
NBT_PAD = NBT + MC


def _mm_body(x_ref, w_ref, o_ref):
    o_ref[...] = jnp.dot(x_ref[...], w_ref[...],
                         preferred_element_type=jnp.float32,
                         precision=lax.Precision.DEFAULT)


def _matmul(x, w):
    k = x.shape[1]
    no = w.shape[1]
    return pl.pallas_call(
        _mm_body,
        grid=(MGRID,),
        in_specs=[pl.BlockSpec((MB, k), lambda i: (i, 0)),
                  pl.BlockSpec((k, no), lambda i: (0, 0))],
        out_specs=pl.BlockSpec((MB, no), lambda i: (i, 0)),
        out_shape=jax.ShapeDtypeStruct((NPAD, no), jnp.float32),
    )(x, w)


def _agg_body(h_hbm, meta_hbm, nrmm_hbm, offs_hbm, sc_hbm, sh_hbm,
              out_hbm,
              offrow_v, meta0_v, meta1_v, nrm0_v, nrm1_v, rows0_v, rows1_v,
              acc_v, scale_v, shift_v, sem):
    wid = lax.axis_index("s") * 2 + lax.axis_index("c")
    pltpu.sync_copy(sc_hbm, scale_v)
    pltpu.sync_copy(sh_hbm, shift_v)
    lane = lax.iota(jnp.int32, LANES)

    def load_chunk(bi, mref, nref):
        pltpu.sync_copy(meta_hbm.at[pl.ds(bi * MROW, MC * MROW)], mref)
        pltpu.sync_copy(nrmm_hbm.at[pl.ds(bi * LANES, MC * LANES)], nref)

    for t in range(2):
        b = wid + t * NWORK

        @pl.when(b < NBLK)
        def _block():
            base_node = b * BLK

            def zero_body(r, carry):
                for v in range(VPR):
                    acc_v[r, pl.ds(v * LANES, LANES)] = jnp.zeros(
                        (LANES,), jnp.float32)
                return carry
            lax.fori_loop(0, BLK, zero_body, 0)

            pltpu.sync_copy(offs_hbm.at[pl.ds(b * LANES, LANES)], offrow_v)
            offrow = offrow_v[...]
            e0 = offrow[0]
            e1 = offrow[1]
            b0 = (e0 // LANES) * LANES
            bi0 = b0 // LANES
            nbat = (e1 - b0 + LANES - 1) // LANES

            @pl.when(nbat > 0)
            def _prologue():
                load_chunk(bi0, meta0_v, nrm0_v)
                pltpu.async_copy(h_hbm.at[meta0_v.at[pl.ds(0, LANES)]],
                                 rows0_v, sem)

            @pl.when(nbat > 1)
            def _prologue2():
                pltpu.async_copy(h_hbm.at[meta0_v.at[pl.ds(MROW, LANES)]],
                                 rows1_v, sem)

            def pair_body(i2, carry):
                for half, rows_cur in ((0, rows0_v), (1, rows1_v)):
                    kk = i2 * 2 + half

                    @pl.when(kk < nbat)
                    def _do():
                        pltpu.make_async_copy(h_hbm.at[pl.ds(0, LANES)],
                                              rows_cur, sem).wait()
                        slab0 = ((kk // MC) % 2) == 0
                        moff = (kk % MC) * MROW
                        noff = (kk % MC) * LANES
                        dstv = jnp.where(
                            slab0,
                            meta0_v[pl.ds(moff + LANES, LANES)],
                            meta1_v[pl.ds(moff + LANES, LANES)])
                        nrmv = jnp.where(slab0,
                                         nrm0_v[pl.ds(noff, LANES)],
                                         nrm1_v[pl.ds(noff, LANES)])
                        base = b0 + kk * LANES
                        gidx = base + lane
                        valid = (gidx >= e0) & (gidx < e1)
                        nrm = jnp.where(valid, nrmv, 0.0)
                        dstl = jnp.where(valid, dstv - base_node, 0)
                        for j in range(LANES):
                            njs = nrm[j]
                            djs = dstl[j]
                            njv = jnp.full((LANES,), njs, jnp.float32)
                            for v in range(VPR):
                                sl = pl.ds(v * LANES, LANES)
                                plsc.addupdate(acc_v.at[djs, sl],
                                               njv * rows_cur[j, sl])

                        kq = kk + 2

                        @pl.when(kq < nbat)
                        def _pf():
                            cq_even = ((kq // MC) % 2) == 0
                            mq = (kq % MC) * MROW

                            @pl.when((kq % MC) == 0)
                            def _chunk():
                                @pl.when(cq_even)
                                def _c0():
                                    load_chunk(bi0 + kq, meta0_v, nrm0_v)

                                @pl.when(jnp.logical_not(cq_even))
                                def _c1():
                                    load_chunk(bi0 + kq, meta1_v, nrm1_v)

                            @pl.when(cq_even)
                            def _g0():
                                pltpu.async_copy(
                                    h_hbm.at[meta0_v.at[pl.ds(mq, LANES)]],
                                    rows_cur, sem)

                            @pl.when(jnp.logical_not(cq_even))
                            def _g1():
                                pltpu.async_copy(
                                    h_hbm.at[meta1_v.at[pl.ds(mq, LANES)]],
                                    rows_cur, sem)
                return carry
            lax.fori_loop(0, (nbat + 1) // 2, pair_body, 0)

            def wb_body(r, carry):
                for v in range(VPR):
                    sl = pl.ds(v * LANES, LANES)
                    y = acc_v[r, sl] * scale_v[sl] + shift_v[sl]
                    acc_v[r, sl] = jnp.maximum(y, 0.0)
                return carry
            lax.fori_loop(0, BLK, wb_body, 0)

            pltpu.sync_copy(acc_v, out_hbm.at[pl.ds(base_node, BLK)])


def _aggregate(h, meta, nrmm, offs, scale, shift):
    mesh = plsc.VectorSubcoreMesh(core_axis_name="c", subcore_axis_name="s")
    kfn = pl.kernel(
        _agg_body,
        out_type=jax.ShapeDtypeStruct((NPAD, H), jnp.float32),
        mesh=mesh,
        scratch_types=[
            pltpu.VMEM((LANES,), jnp.int32),
            pltpu.VMEM((MC * MROW,), jnp.int32),
            pltpu.VMEM((MC * MROW,), jnp.int32),
            pltpu.VMEM((MC * LANES,), jnp.float32),
            pltpu.VMEM((MC * LANES,), jnp.float32),
            pltpu.VMEM((LANES, H), jnp.float32),
            pltpu.VMEM((LANES, H), jnp.float32),
            pltpu.VMEM((BLK, H), jnp.float32),
            pltpu.VMEM((H,), jnp.float32),
            pltpu.VMEM((H,), jnp.float32),
            pltpu.SemaphoreType.DMA,
        ],
    )
    return kfn(h, meta, nrmm, offs, scale, shift)


def _pool_body(a_ref, b_ref, wl_ref, bl_ref, o_ref, sums, cnt):
    i = pl.program_id(0)

    @pl.when(i == 0)
    def _init():
        sums[...] = jnp.zeros_like(sums)
        cnt[...] = jnp.zeros_like(cnt)

    bv = b_ref[0]
    gids = lax.broadcasted_iota(jnp.int32, (G, MB), 0)
    m = jnp.where(bv == gids, 1.0, 0.0)
    sums[...] += jnp.dot(m, a_ref[...],
                         preferred_element_type=jnp.float32,
                         precision=lax.Precision.HIGHEST)
    cnt[...] += jnp.broadcast_to(jnp.sum(m, axis=1, keepdims=True), (G, 128))

    @pl.when(i == pl.num_programs(0) - 1)
    def _fin():
        c = jnp.maximum(cnt[:, 0:1], 1.0)
        pooled = sums[...] / c
        mu = jnp.mean(pooled, axis=-1, keepdims=True)
        var = jnp.mean((pooled - mu) ** 2, axis=-1, keepdims=True)
        ln = (pooled - mu) * lax.rsqrt(var + 1e-5)
        o_ref[...] = jnp.dot(ln, wl_ref[...],
                             preferred_element_type=jnp.float32,
                             precision=lax.Precision.DEFAULT) + bl_ref[...]


def _pool(a, batch3, wlp, blr):
    return pl.pallas_call(
        _pool_body,
        grid=(MGRID,),
        in_specs=[pl.BlockSpec((MB, H), lambda i: (i, 0)),
                  pl.BlockSpec((1, 1, MB), lambda i: (i, 0, 0)),
                  pl.BlockSpec((H, 128), lambda i: (0, 0)),
                  pl.BlockSpec((1, 128), lambda i: (0, 0))],
        out_specs=pl.BlockSpec((G, 128), lambda i: (0, 0)),
        out_shape=jax.ShapeDtypeStruct((G, 128), jnp.float32),
        scratch_shapes=[pltpu.VMEM((G, H), jnp.float32),
                        pltpu.VMEM((G, 128), jnp.float32)],
    )(a, batch3, wlp, blr)


def kernel(x, edge_index, batch, W1, b1, W2, b2, W3, b3,
           bn1_g, bn1_b, bn1_m, bn1_v,
           bn2_g, bn2_b, bn2_m, bn2_v,
           bn3_g, bn3_b, bn3_m, bn3_v,
           Wl, bl):
    f32 = jnp.float32
    ar = jnp.arange(N, dtype=jnp.int32)
    src_a = jnp.concatenate([edge_index[0], ar])
    dst_a = jnp.concatenate([edge_index[1], ar])
    perm = jnp.argsort(dst_a)
    src_s = src_a[perm]
    dst_s = dst_a[perm]
    left = jnp.searchsorted(dst_s, ar, side="left")
    right = jnp.searchsorted(dst_s, ar, side="right")
    deg = (right - left).astype(f32)
    dinv = lax.rsqrt(deg)
    nrm_s = dinv[src_s] * dinv[dst_s]
    bounds = jnp.arange(NBLK + 1, dtype=jnp.int32) * BLK
    off = jnp.searchsorted(dst_s, bounds, side="left").astype(jnp.int32)
    offs = (jnp.zeros((NBLK, LANES), jnp.int32)
            .at[:, 0].set(off[:-1])
            .at[:, 1].set(off[1:])).reshape(-1)
    meta = (jnp.zeros((NBT_PAD, 2, LANES), jnp.int32)
            .at[:NBT, 0, :].set(src_s.reshape(NBT, LANES))
            .at[:NBT, 1, :].set(dst_s.reshape(NBT, LANES))).reshape(-1)
    nrmm = (jnp.zeros((NBT_PAD, LANES), jnp.float32)
            .at[:NBT, :].set(nrm_s.reshape(NBT, LANES))).reshape(-1)

    def fold(g, bta, m, v, b_lin):
        sc = g * lax.rsqrt(v + 1e-5)
        return sc, (b_lin - m) * sc + bta

    sc1, sh1 = fold(bn1_g, bn1_b, bn1_m, bn1_v, b1)
    sc2, sh2 = fold(bn2_g, bn2_b, bn2_m, bn2_v, b2)
    sc3, sh3 = fold(bn3_g, bn3_b, bn3_m, bn3_v, b3)

    xp = jnp.pad(x, ((0, NPAD - N), (0, 0)))
    h = _matmul(xp, W1)
    a = _aggregate(h, meta, nrmm, offs, sc1, sh1)
    h = _matmul(a, W2)
    a = _aggregate(h, meta, nrmm, offs, sc2, sh2)
    h = _matmul(a, W3)
    a = _aggregate(h, meta, nrmm, offs, sc3, sh3)

    batch3 = jnp.pad(batch, (0, NPAD - N), constant_values=G)
    batch3 = batch3.reshape(MGRID, 1, MB)
    wlp = jnp.pad(Wl, ((0, 0), (0, 127)))
    blr = jnp.broadcast_to(bl.reshape(1, 1), (1, 128))
    out = _pool(a, batch3, wlp, blr)
    return out[:, 0:1]


# speedup vs baseline: 1.7753x; 1.0098x over previous
import functools

import jax
import jax.numpy as jnp
from jax import lax
from jax.experimental import pallas as pl
from jax.experimental.pallas import tpu as pltpu
from jax.experimental.pallas import tpu_sc as plsc

N = 10000
E = 160000
H = 512
G = 64
LANES = 16
NWORK = 32
BLK = 160
NBLK = 63
NPAD = NBLK * BLK
EA = E + N
VPR = H // LANES
MGRID = 9
MB = NPAD // MGRID
NBT = EA // LANES
MC = 32
MROW = 2 * LANES
NBT_PAD = NBT + MC


def _mm_body(x_ref, w_ref, o_ref):
    o_ref[...] = jnp.dot(x_ref[...], w_ref[...],
                         preferred_element_type=jnp.float32,
                         precision=lax.Precision.DEFAULT)


def _matmul(x, w):
    k = x.shape[1]
    no = w.shape[1]
    return pl.pallas_call(
        _mm_body,
        grid=(MGRID,),
        in_specs=[pl.BlockSpec((MB, k), lambda i: (i, 0)),
                  pl.BlockSpec((k, no), lambda i: (0, 0))],
        out_specs=pl.BlockSpec((MB, no), lambda i: (i, 0)),
        out_shape=jax.ShapeDtypeStruct((NPAD, no), jnp.float32),
    )(x, w)


NSLOT = 2


def _agg_body(h_hbm, meta_hbm, nrmm_hbm, offs_hbm, sc_hbm, sh_hbm,
              out_hbm,
              offrow_v, meta0_v, meta1_v, nrm0_v, nrm1_v,
              rows0_v, rows1_v,
              acc_v, scale_v, shift_v, sem):
    wid = lax.axis_index("s") * 2 + lax.axis_index("c")
    pltpu.sync_copy(sc_hbm, scale_v)
    pltpu.sync_copy(sh_hbm, shift_v)
    lane = lax.iota(jnp.int32, LANES)

    def load_chunk(bi, mref, nref):
        pltpu.sync_copy(meta_hbm.at[pl.ds(bi * MROW, MC * MROW)], mref)
        pltpu.sync_copy(nrmm_hbm.at[pl.ds(bi * LANES, MC * LANES)], nref)

    for t in range(2):
        b = wid + t * NWORK

        @pl.when(b < NBLK)
        def _block():
            base_node = b * BLK

            def zero_body(r, carry):
                for v in range(VPR):
                    acc_v[r, pl.ds(v * LANES, LANES)] = jnp.zeros(
                        (LANES,), jnp.float32)
                return carry
            lax.fori_loop(0, BLK, zero_body, 0)

            pltpu.sync_copy(offs_hbm.at[pl.ds(b * LANES, LANES)], offrow_v)
            offrow = offrow_v[...]
            e0 = offrow[0]
            e1 = offrow[1]
            b0 = (e0 // LANES) * LANES
            bi0 = b0 // LANES
            nbat = (e1 - b0 + LANES - 1) // LANES

            slot_refs = (rows0_v, rows1_v)

            @pl.when(nbat > 0)
            def _prologue():
                load_chunk(bi0, meta0_v, nrm0_v)

            for p in range(NSLOT):
                @pl.when(nbat > p)
                def _pro(p=p):
                    pltpu.async_copy(
                        h_hbm.at[meta0_v.at[pl.ds(p * MROW, LANES)]],
                        slot_refs[p], sem)

            def pair_body(i2, carry):
                for half in range(NSLOT):
                    rows_cur = slot_refs[half]
                    kk = i2 * NSLOT + half

                    @pl.when(kk < nbat)
                    def _do():
                        pltpu.make_async_copy(h_hbm.at[pl.ds(0, LANES)],
                                              rows_cur, sem).wait()
                        slab0 = ((kk // MC) % 2) == 0
                        moff = (kk % MC) * MROW
                        noff = (kk % MC) * LANES
                        dstv = jnp.where(
                            slab0,
                            meta0_v[pl.ds(moff + LANES, LANES)],
                            meta1_v[pl.ds(moff + LANES, LANES)])
                        nrmv = jnp.where(slab0,
                                         nrm0_v[pl.ds(noff, LANES)],
                                         nrm1_v[pl.ds(noff, LANES)])
                        base = b0 + kk * LANES
                        gidx = base + lane
                        valid = (gidx >= e0) & (gidx < e1)
                        nrm = jnp.where(valid, nrmv, 0.0)
                        dstl = jnp.where(valid, dstv - base_node, 0)
                        for j in range(LANES):
                            njs = nrm[j]
                            djs = dstl[j]
                            njv = jnp.full((LANES,), njs, jnp.float32)
                            for v in range(VPR):
                                sl = pl.ds(v * LANES, LANES)
                                plsc.addupdate(acc_v.at[djs, sl],
                                               njv * rows_cur[j, sl])

                        kq = kk + NSLOT

                        @pl.when(kq < nbat)
                        def _pf():
                            cq_even = ((kq // MC) % 2) == 0
                            mq = (kq % MC) * MROW

                            @pl.when((kq % MC) == 0)
                            def _chunk():
                                @pl.when(cq_even)
                                def _c0():
                                    load_chunk(bi0 + kq, meta0_v, nrm0_v)

                                @pl.when(jnp.logical_not(cq_even))
                                def _c1():
                                    load_chunk(bi0 + kq, meta1_v, nrm1_v)

                            @pl.when(cq_even)
                            def _g0():
                                pltpu.async_copy(
                                    h_hbm.at[meta0_v.at[pl.ds(mq, LANES)]],
                                    rows_cur, sem)

                            @pl.when(jnp.logical_not(cq_even))
                            def _g1():
                                pltpu.async_copy(
                                    h_hbm.at[meta1_v.at[pl.ds(mq, LANES)]],
                                    rows_cur, sem)
                return carry
            lax.fori_loop(0, (nbat + NSLOT - 1) // NSLOT, pair_body, 0)

            def wb_body(r, carry):
                for v in range(VPR):
                    sl = pl.ds(v * LANES, LANES)
                    y = acc_v[r, sl] * scale_v[sl] + shift_v[sl]
                    acc_v[r, sl] = jnp.maximum(y, 0.0)
                return carry
            lax.fori_loop(0, BLK, wb_body, 0)

            pltpu.sync_copy(acc_v, out_hbm.at[pl.ds(base_node, BLK)])


def _aggregate(h, meta, nrmm, offs, scale, shift):
    mesh = plsc.VectorSubcoreMesh(core_axis_name="c", subcore_axis_name="s")
    kfn = pl.kernel(
        _agg_body,
        out_type=jax.ShapeDtypeStruct((NPAD, H), jnp.float32),
        mesh=mesh,
        scratch_types=[
            pltpu.VMEM((LANES,), jnp.int32),
            pltpu.VMEM((MC * MROW,), jnp.int32),
            pltpu.VMEM((MC * MROW,), jnp.int32),
            pltpu.VMEM((MC * LANES,), jnp.float32),
            pltpu.VMEM((MC * LANES,), jnp.float32),
            pltpu.VMEM((LANES, H), jnp.float32),
            pltpu.VMEM((LANES, H), jnp.float32),
            pltpu.VMEM((BLK, H), jnp.float32),
            pltpu.VMEM((H,), jnp.float32),
            pltpu.VMEM((H,), jnp.float32),
            pltpu.SemaphoreType.DMA,
        ],
    )
    return kfn(h, meta, nrmm, offs, scale, shift)


def _pool_body(a_ref, b_ref, wl_ref, bl_ref, o_ref, sums, cnt):
    i = pl.program_id(0)

    @pl.when(i == 0)
    def _init():
        sums[...] = jnp.zeros_like(sums)
        cnt[...] = jnp.zeros_like(cnt)

    bv = b_ref[0]
    gids = lax.broadcasted_iota(jnp.int32, (G, MB), 0)
    m = jnp.where(bv == gids, 1.0, 0.0)
    sums[...] += jnp.dot(m, a_ref[...],
                         preferred_element_type=jnp.float32,
                         precision=lax.Precision.HIGHEST)
    cnt[...] += jnp.broadcast_to(jnp.sum(m, axis=1, keepdims=True), (G, 128))

    @pl.when(i == pl.num_programs(0) - 1)
    def _fin():
        c = jnp.maximum(cnt[:, 0:1], 1.0)
        pooled = sums[...] / c
        mu = jnp.mean(pooled, axis=-1, keepdims=True)
        var = jnp.mean((pooled - mu) ** 2, axis=-1, keepdims=True)
        ln = (pooled - mu) * lax.rsqrt(var + 1e-5)
        o_ref[...] = jnp.dot(ln, wl_ref[...],
                             preferred_element_type=jnp.float32,
                             precision=lax.Precision.DEFAULT) + bl_ref[...]


def _pool(a, batch3, wlp, blr):
    return pl.pallas_call(
        _pool_body,
        grid=(MGRID,),
        in_specs=[pl.BlockSpec((MB, H), lambda i: (i, 0)),
                  pl.BlockSpec((1, 1, MB), lambda i: (i, 0, 0)),
                  pl.BlockSpec((H, 128), lambda i: (0, 0)),
                  pl.BlockSpec((1, 128), lambda i: (0, 0))],
        out_specs=pl.BlockSpec((G, 128), lambda i: (0, 0)),
        out_shape=jax.ShapeDtypeStruct((G, 128), jnp.float32),
        scratch_shapes=[pltpu.VMEM((G, H), jnp.float32),
                        pltpu.VMEM((G, 128), jnp.float32)],
    )(a, batch3, wlp, blr)


def kernel(x, edge_index, batch, W1, b1, W2, b2, W3, b3,
           bn1_g, bn1_b, bn1_m, bn1_v,
           bn2_g, bn2_b, bn2_m, bn2_v,
           bn3_g, bn3_b, bn3_m, bn3_v,
           Wl, bl):
    f32 = jnp.float32
    ar = jnp.arange(N, dtype=jnp.int32)
    src_a = jnp.concatenate([edge_index[0], ar])
    dst_a = jnp.concatenate([edge_index[1], ar])
    packed = jnp.sort(dst_a * 16384 + src_a)
    src_s = packed & 16383
    dst_s = packed >> 14
    left = jnp.searchsorted(dst_s, ar, side="left")
    right = jnp.searchsorted(dst_s, ar, side="right")
    deg = (right - left).astype(f32)
    dinv = lax.rsqrt(deg)
    nrm_s = dinv[src_s] * dinv[dst_s]
    bounds = jnp.arange(NBLK + 1, dtype=jnp.int32) * BLK
    off = jnp.searchsorted(dst_s, bounds, side="left").astype(jnp.int32)
    offs = (jnp.zeros((NBLK, LANES), jnp.int32)
            .at[:, 0].set(off[:-1])
            .at[:, 1].set(off[1:])).reshape(-1)
    meta = (jnp.zeros((NBT_PAD, 2, LANES), jnp.int32)
            .at[:NBT, 0, :].set(src_s.reshape(NBT, LANES))
            .at[:NBT, 1, :].set(dst_s.reshape(NBT, LANES))).reshape(-1)
    nrmm = (jnp.zeros((NBT_PAD, LANES), jnp.float32)
            .at[:NBT, :].set(nrm_s.reshape(NBT, LANES))).reshape(-1)

    def fold(g, bta, m, v, b_lin):
        sc = g * lax.rsqrt(v + 1e-5)
        return sc, (b_lin - m) * sc + bta

    sc1, sh1 = fold(bn1_g, bn1_b, bn1_m, bn1_v, b1)
    sc2, sh2 = fold(bn2_g, bn2_b, bn2_m, bn2_v, b2)
    sc3, sh3 = fold(bn3_g, bn3_b, bn3_m, bn3_v, b3)

    xp = jnp.pad(x, ((0, NPAD - N), (0, 0)))
    h = _matmul(xp, W1)
    a = _aggregate(h, meta, nrmm, offs, sc1, sh1)
    h = _matmul(a, W2)
    a = _aggregate(h, meta, nrmm, offs, sc2, sh2)
    h = _matmul(a, W3)
    a = _aggregate(h, meta, nrmm, offs, sc3, sh3)

    batch3 = jnp.pad(batch, (0, NPAD - N), constant_values=G)
    batch3 = batch3.reshape(MGRID, 1, MB)
    wlp = jnp.pad(Wl, ((0, 0), (0, 127)))
    blr = jnp.broadcast_to(bl.reshape(1, 1), (1, 128))
    out = _pool(a, batch3, wlp, blr)
    return out[:, 0:1]
